# Initial kernel scaffold; baseline (speedup 1.0000x reference)
import jax
import jax.numpy as jnp
from jax.experimental import pallas as pl


def kernel(edge_index, emb, W1, b1, W2, b2, Wfc, bfc):
    raise NotImplementedError("write your pallas kernel here")


# speedup vs baseline: 7.3608x; 7.3608x over previous
import functools

import jax
import jax.numpy as jnp
from jax import lax
from jax.experimental import pallas as pl
from jax.experimental.pallas import tpu as pltpu
from jax.experimental.pallas import tpu_sc as plsc

N = 50000
E = 800000
NC, NS = 2, 16
NW = NC * NS
BL = 128
NBT = E // BL
NB_HI = 196
NB_LO = 195
HI = NBT - NW * NB_LO
NPAD = 51200
SLICE = NPAD // NS
ZB = SLICE // BL
CW = 16
RB = 1024
GRID = 49

_MESH = dict(core_axis_name="c", subcore_axis_name="s", num_cores=NC,
             num_subcores=NS)
_F32 = jnp.float32
_SC_PARAMS = pltpu.CompilerParams(needs_layout_passes=False,
                                  use_tc_tiling_on_sc=False)


def _tile_range(wid):
    row0 = NB_LO * wid + jnp.minimum(wid, HI)
    nblk = jnp.where(wid < HI, NB_HI, NB_LO)
    return row0, nblk


@functools.partial(
    pl.kernel,
    out_type=jax.ShapeDtypeStruct((NW, NPAD), _F32),
    mesh=plsc.VectorSubcoreMesh(**_MESH),
    compiler_params=_SC_PARAMS,
    scratch_types=[
        pltpu.VMEM((NB_HI * BL,), jnp.int32),
        pltpu.VMEM((NPAD,), _F32),
    ],
)
def _deg_kernel(dst_hbm, zeros_hbm, out_hbm, dst_v, deg_v):
    wid = lax.axis_index("c") * NS + lax.axis_index("s")
    row0, nblk = _tile_range(wid)
    e0 = row0 * BL

    @pl.when(wid < HI)
    def _():
        pltpu.sync_copy(dst_hbm.at[pl.ds(e0, NB_HI * BL)], dst_v)

    @pl.when(wid >= HI)
    def _():
        pltpu.sync_copy(dst_hbm.at[pl.ds(e0, NB_LO * BL)],
                        dst_v.at[pl.ds(0, NB_LO * BL)])

    pltpu.sync_copy(zeros_hbm, deg_v)
    ones = jnp.ones((16,), _F32)

    def body(q, carry):
        idx = dst_v[pl.ds(q * 16, 16)]
        plsc.addupdate_scatter(deg_v, [idx], ones)
        return carry

    lax.fori_loop(0, nblk * (BL // 16), body, 0)
    pltpu.sync_copy(deg_v, out_hbm.at[wid])


def _make_agg_kernel(n_chunks):
    @functools.partial(
        pl.kernel,
        out_type=jax.ShapeDtypeStruct((n_chunks, NC, NPAD, CW), _F32),
        mesh=plsc.VectorSubcoreMesh(**_MESH),
        compiler_params=_SC_PARAMS,
        scratch_types=[
            pltpu.VMEM((NB_HI, BL), jnp.int32),
            pltpu.VMEM((NB_HI, BL), jnp.int32),
            pltpu.VMEM((BL, CW), _F32),
            pltpu.VMEM((BL, CW), _F32),
            pltpu.VMEM_SHARED((NPAD, CW), _F32),
            pltpu.SemaphoreType.DMA,
        ],
    )
    def agg_kernel(src_hbm, dst_hbm, tbl_hbm, zeros_hbm, out_hbm,
                   src_v, dst_v, rows_v, zbuf, agg_sh, sem):
        cid = lax.axis_index("c")
        sid = lax.axis_index("s")
        wid = cid * NS + sid
        base = sid * SLICE
        row0, nblk = _tile_range(wid)

        @pl.when(wid < HI)
        def _():
            pltpu.sync_copy(src_hbm.at[pl.ds(row0, NB_HI)], src_v)
            pltpu.sync_copy(dst_hbm.at[pl.ds(row0, NB_HI)], dst_v)

        @pl.when(wid >= HI)
        def _():
            pltpu.sync_copy(src_hbm.at[pl.ds(row0, NB_LO)],
                            src_v.at[pl.ds(0, NB_LO)])
            pltpu.sync_copy(dst_hbm.at[pl.ds(row0, NB_LO)],
                            dst_v.at[pl.ds(0, NB_LO)])

        pltpu.sync_copy(zeros_hbm, zbuf)

        for t in range(n_chunks):
            def zbody(z, carry):
                pltpu.sync_copy(zbuf, agg_sh.at[pl.ds(base + z * BL, BL)])
                return carry

            lax.fori_loop(0, ZB, zbody, 0)
            plsc.subcore_barrier()

            tbl = tbl_hbm.at[t]

            def gbody(j, carry):
                pltpu.async_copy(tbl.at[src_v.at[j]], rows_v, sem).wait()
                pltpu.sync_copy(rows_v, agg_sh.at[dst_v.at[j]], add=True)
                return carry

            lax.fori_loop(0, nblk, gbody, 0)
            plsc.subcore_barrier()

            def obody(z, carry):
                pltpu.sync_copy(agg_sh.at[pl.ds(base + z * BL, BL)], rows_v)
                pltpu.sync_copy(
                    rows_v, out_hbm.at[t, cid, pl.ds(base + z * BL, BL)])
                return carry

            lax.fori_loop(0, ZB, obody, 0)
            plsc.subcore_barrier()

    return agg_kernel


_agg8 = _make_agg_kernel(8)
_agg4 = _make_agg_kernel(4)


_HI_PREC = jax.lax.Precision.HIGHEST


def _tc1_body(deg_ref, emb_ref, w_ref, dis_ref, hs_ref):
    d = jnp.sum(deg_ref[...], axis=1, keepdims=True) + 1.0
    dis = jax.lax.rsqrt(d)
    h = jnp.dot(emb_ref[...], w_ref[...], preferred_element_type=_F32,
                precision=_HI_PREC)
    hs = h * dis
    dis_ref[...] = dis
    for c in range(8):
        hs_ref[c] = hs[:, c * CW:(c + 1) * CW]


def _tc1(degT, emb, W1):
    return pl.pallas_call(
        _tc1_body,
        grid=(GRID,),
        in_specs=[
            pl.BlockSpec((RB, NW), lambda i: (i, 0)),
            pl.BlockSpec((RB, 64), lambda i: (i, 0)),
            pl.BlockSpec((64, 128), lambda i: (0, 0)),
        ],
        out_specs=[
            pl.BlockSpec((RB, 1), lambda i: (i, 0)),
            pl.BlockSpec((8, RB, CW), lambda i: (0, i, 0)),
        ],
        out_shape=[
            jax.ShapeDtypeStruct((N, 1), _F32),
            jax.ShapeDtypeStruct((8, N, CW), _F32),
        ],
    )(degT, emb, W1)


def _tc2_body(p_ref, hs_ref, dis_ref, b_ref, w_ref, out_ref):
    dis = dis_ref[...]
    acc = jnp.zeros((RB, 64), _F32)
    for c in range(8):
        agg = p_ref[c, 0] + p_ref[c, 1] + hs_ref[c]
        x = jnp.maximum(dis * agg + b_ref[c], 0.0)
        acc = acc + jnp.dot(x, w_ref[c * CW:(c + 1) * CW, :],
                            preferred_element_type=_F32, precision=_HI_PREC)
    hs2 = acc * dis
    for d in range(4):
        out_ref[d] = hs2[:, d * CW:(d + 1) * CW]


def _tc2(p1, hs1, dis, b1r, W2):
    return pl.pallas_call(
        _tc2_body,
        grid=(GRID,),
        in_specs=[
            pl.BlockSpec((8, NC, RB, CW), lambda i: (0, 0, i, 0)),
            pl.BlockSpec((8, RB, CW), lambda i: (0, i, 0)),
            pl.BlockSpec((RB, 1), lambda i: (i, 0)),
            pl.BlockSpec((8, 1, CW), lambda i: (0, 0, 0)),
            pl.BlockSpec((128, 64), lambda i: (0, 0)),
        ],
        out_specs=pl.BlockSpec((4, RB, CW), lambda i: (0, i, 0)),
        out_shape=jax.ShapeDtypeStruct((4, N, CW), _F32),
    )(p1, hs1, dis, b1r, W2)


def _tc3_body(p_ref, hs_ref, dis_ref, b_ref, w_ref, bfc_ref, out_ref):
    dis = dis_ref[...]
    acc = jnp.zeros((RB, 1), _F32)
    for c in range(4):
        agg = p_ref[c, 0] + p_ref[c, 1] + hs_ref[c]
        x = jnp.maximum(dis * agg + b_ref[c], 0.0)
        acc = acc + jnp.dot(x, w_ref[c], preferred_element_type=_F32,
                            precision=_HI_PREC)
    out_ref[...] = acc + bfc_ref[0, 0]


def _tc3(p2, hs2, dis, b2r, Wfcr, bfcr):
    return pl.pallas_call(
        _tc3_body,
        grid=(GRID,),
        in_specs=[
            pl.BlockSpec((4, NC, RB, CW), lambda i: (0, 0, i, 0)),
            pl.BlockSpec((4, RB, CW), lambda i: (0, i, 0)),
            pl.BlockSpec((RB, 1), lambda i: (i, 0)),
            pl.BlockSpec((4, 1, CW), lambda i: (0, 0, 0)),
            pl.BlockSpec((4, CW, 1), lambda i: (0, 0, 0)),
            pl.BlockSpec((1, 1), lambda i: (0, 0)),
        ],
        out_specs=pl.BlockSpec((RB, 1), lambda i: (i, 0)),
        out_shape=jax.ShapeDtypeStruct((N, 1), _F32),
    )(p2, hs2, dis, b2r, Wfcr, bfcr)


def kernel(edge_index, emb, W1, b1, W2, b2, Wfc, bfc):
    src_b = edge_index[0].reshape(NBT, BL)
    dst_b = edge_index[1].reshape(NBT, BL)
    dst_f = edge_index[1]
    zeros_n = jnp.zeros((NPAD,), _F32)
    zeros_t = jnp.zeros((BL, CW), _F32)

    deg_parts = _deg_kernel(dst_f, zeros_n)
    degT = deg_parts.T
    dis, hs1 = _tc1(degT, emb, W1)
    p1 = _agg8(src_b, dst_b, hs1, zeros_t)
    hs2 = _tc2(p1, hs1, dis, b1.reshape(8, 1, CW), W2)
    p2 = _agg4(src_b, dst_b, hs2, zeros_t)
    return _tc3(p2, hs2, dis, b2.reshape(4, 1, CW),
                Wfc.reshape(4, CW, 1), bfc.reshape(1, 1))


# speedup vs baseline: 9.1229x; 1.2394x over previous
import functools

import jax
import jax.numpy as jnp
from jax import lax
from jax.experimental import pallas as pl
from jax.experimental.pallas import tpu as pltpu
from jax.experimental.pallas import tpu_sc as plsc

N = 50000
E = 800000
NC, NS = 2, 16
NW = NC * NS
BL = 128
NBT = E // BL
NB_HI = 196
NB_LO = 195
HI = NBT - NW * NB_LO
NPAD = 51200
SLICE = NPAD // NS
ZB = SLICE // BL
CW = 16
RB = 1024
GRID = 49

_MESH = dict(core_axis_name="c", subcore_axis_name="s", num_cores=NC,
             num_subcores=NS)
_F32 = jnp.float32
_SC_PARAMS = pltpu.CompilerParams(needs_layout_passes=False,
                                  use_tc_tiling_on_sc=False)


def _tile_range(wid):
    row0 = NB_LO * wid + jnp.minimum(wid, HI)
    nblk = jnp.where(wid < HI, NB_HI, NB_LO)
    return row0, nblk


@functools.partial(
    pl.kernel,
    out_type=jax.ShapeDtypeStruct((NW, NPAD), _F32),
    mesh=plsc.VectorSubcoreMesh(**_MESH),
    compiler_params=_SC_PARAMS,
    scratch_types=[
        pltpu.VMEM((NB_HI * BL,), jnp.int32),
        pltpu.VMEM((NPAD,), _F32),
    ],
)
def _deg_kernel(dst_hbm, zeros_hbm, out_hbm, dst_v, deg_v):
    wid = lax.axis_index("c") * NS + lax.axis_index("s")
    row0, nblk = _tile_range(wid)
    e0 = row0 * BL

    @pl.when(wid < HI)
    def _():
        pltpu.sync_copy(dst_hbm.at[pl.ds(e0, NB_HI * BL)], dst_v)

    @pl.when(wid >= HI)
    def _():
        pltpu.sync_copy(dst_hbm.at[pl.ds(e0, NB_LO * BL)],
                        dst_v.at[pl.ds(0, NB_LO * BL)])

    pltpu.sync_copy(zeros_hbm, deg_v)
    ones = jnp.ones((16,), _F32)

    def body(q, carry):
        idx = dst_v[pl.ds(q * 16, 16)]
        plsc.addupdate_scatter(deg_v, [idx], ones)
        return carry

    lax.fori_loop(0, nblk * (BL // 16), body, 0)
    pltpu.sync_copy(deg_v, out_hbm.at[wid])


NBUF = 8
NB_PAD = 200
NGRP = NB_PAD // NBUF
PAD_DST = 50688


def _make_agg_kernel(n_chunks):
    @functools.partial(
        pl.kernel,
        out_type=jax.ShapeDtypeStruct((n_chunks, NC, NPAD, CW), _F32),
        mesh=plsc.VectorSubcoreMesh(**_MESH),
        compiler_params=_SC_PARAMS,
        scratch_types=[
            pltpu.VMEM((NB_PAD, BL), jnp.int32),
            pltpu.VMEM((NB_PAD, BL), jnp.int32),
            pltpu.VMEM((NBUF, BL, CW), _F32),
            pltpu.VMEM((BL, CW), _F32),
            pltpu.VMEM_SHARED((NPAD, CW), _F32),
            pltpu.SemaphoreType.DMA((NBUF,)),
            pltpu.SemaphoreType.DMA((NBUF,)),
        ],
    )
    def agg_kernel(src_hbm, dst_hbm, tbl_hbm, zeros_hbm, out_hbm,
                   src_v, dst_v, rows_v, zbuf, agg_sh, gsem, ssem):
        cid = lax.axis_index("c")
        sid = lax.axis_index("s")
        wid = cid * NS + sid
        base = sid * SLICE
        row0, _ = _tile_range(wid)

        @pl.when(wid < HI)
        def _():
            pltpu.sync_copy(src_hbm.at[pl.ds(row0, NB_HI)],
                            src_v.at[pl.ds(0, NB_HI)])
            pltpu.sync_copy(dst_hbm.at[pl.ds(row0, NB_HI)],
                            dst_v.at[pl.ds(0, NB_HI)])

        @pl.when(wid >= HI)
        def _():
            pltpu.sync_copy(src_hbm.at[pl.ds(row0, NB_LO)],
                            src_v.at[pl.ds(0, NB_LO)])
            pltpu.sync_copy(dst_hbm.at[pl.ds(row0, NB_LO)],
                            dst_v.at[pl.ds(0, NB_LO)])

        pltpu.sync_copy(zeros_hbm, zbuf)

        z16 = jnp.zeros((16,), jnp.int32)
        d16 = jnp.full((16,), PAD_DST, jnp.int32)
        for r in range(NB_HI, NB_PAD):
            for k in range(BL // 16):
                src_v[r, pl.ds(k * 16, 16)] = z16
                dst_v[r, pl.ds(k * 16, 16)] = d16

        @pl.when(wid >= HI)
        def _():
            for k in range(BL // 16):
                src_v[NB_LO, pl.ds(k * 16, 16)] = z16
                dst_v[NB_LO, pl.ds(k * 16, 16)] = d16

        for t in range(n_chunks):
            def zbody(z, carry):
                pltpu.sync_copy(zbuf, agg_sh.at[pl.ds(base + z * BL, BL)])
                return carry

            lax.fori_loop(0, ZB, zbody, 0)
            plsc.subcore_barrier()

            tbl = tbl_hbm.at[t]

            def _gather(j, b):
                pltpu.async_copy(tbl.at[src_v.at[j]], rows_v.at[b],
                                 gsem.at[b])

            def _gather_wait(j, b):
                pltpu.make_async_copy(tbl.at[src_v.at[j]], rows_v.at[b],
                                      gsem.at[b]).wait()

            def _scatter(j, b):
                pltpu.async_copy(rows_v.at[b], agg_sh.at[dst_v.at[j]],
                                 ssem.at[b], add=True)

            def _scatter_wait(j, b):
                pltpu.make_async_copy(rows_v.at[b], agg_sh.at[dst_v.at[j]],
                                      ssem.at[b]).wait()

            for b in range(NBUF):
                _gather(b, b)

            def gbody(g, carry):
                for b in range(NBUF):
                    j = g * NBUF + b
                    _gather_wait(j, b)
                    _scatter(j, b)
                for b in range(NBUF):
                    j = g * NBUF + b
                    _scatter_wait(j, b)

                    @pl.when(g < NGRP - 1)
                    def _():
                        _gather(j + NBUF, b)

                return carry

            lax.fori_loop(0, NGRP, gbody, 0)
            plsc.subcore_barrier()

            pltpu.sync_copy(agg_sh.at[pl.ds(base, SLICE)],
                            out_hbm.at[t, cid, pl.ds(base, SLICE)])
            plsc.subcore_barrier()

    return agg_kernel


_agg8 = _make_agg_kernel(8)
_agg4 = _make_agg_kernel(4)


_HI_PREC = jax.lax.Precision.HIGHEST


def _tc1_body(deg_ref, emb_ref, w_ref, dis_ref, hs_ref):
    d = jnp.sum(deg_ref[...], axis=1, keepdims=True) + 1.0
    dis = jax.lax.rsqrt(d)
    h = jnp.dot(emb_ref[...], w_ref[...], preferred_element_type=_F32,
                precision=_HI_PREC)
    hs = h * dis
    dis_ref[...] = dis
    for c in range(8):
        hs_ref[c] = hs[:, c * CW:(c + 1) * CW]


def _tc1(degT, emb, W1):
    return pl.pallas_call(
        _tc1_body,
        grid=(GRID,),
        in_specs=[
            pl.BlockSpec((RB, NW), lambda i: (i, 0)),
            pl.BlockSpec((RB, 64), lambda i: (i, 0)),
            pl.BlockSpec((64, 128), lambda i: (0, 0)),
        ],
        out_specs=[
            pl.BlockSpec((RB, 1), lambda i: (i, 0)),
            pl.BlockSpec((8, RB, CW), lambda i: (0, i, 0)),
        ],
        out_shape=[
            jax.ShapeDtypeStruct((N, 1), _F32),
            jax.ShapeDtypeStruct((8, N, CW), _F32),
        ],
    )(degT, emb, W1)


def _tc2_body(p_ref, hs_ref, dis_ref, b_ref, w_ref, out_ref):
    dis = dis_ref[...]
    acc = jnp.zeros((RB, 64), _F32)
    for c in range(8):
        agg = p_ref[c, 0] + p_ref[c, 1] + hs_ref[c]
        x = jnp.maximum(dis * agg + b_ref[c], 0.0)
        acc = acc + jnp.dot(x, w_ref[c * CW:(c + 1) * CW, :],
                            preferred_element_type=_F32, precision=_HI_PREC)
    hs2 = acc * dis
    for d in range(4):
        out_ref[d] = hs2[:, d * CW:(d + 1) * CW]


def _tc2(p1, hs1, dis, b1r, W2):
    return pl.pallas_call(
        _tc2_body,
        grid=(GRID,),
        in_specs=[
            pl.BlockSpec((8, NC, RB, CW), lambda i: (0, 0, i, 0)),
            pl.BlockSpec((8, RB, CW), lambda i: (0, i, 0)),
            pl.BlockSpec((RB, 1), lambda i: (i, 0)),
            pl.BlockSpec((8, 1, CW), lambda i: (0, 0, 0)),
            pl.BlockSpec((128, 64), lambda i: (0, 0)),
        ],
        out_specs=pl.BlockSpec((4, RB, CW), lambda i: (0, i, 0)),
        out_shape=jax.ShapeDtypeStruct((4, N, CW), _F32),
    )(p1, hs1, dis, b1r, W2)


def _tc3_body(p_ref, hs_ref, dis_ref, b_ref, w_ref, bfc_ref, out_ref):
    dis = dis_ref[...]
    acc = jnp.zeros((RB, 1), _F32)
    for c in range(4):
        agg = p_ref[c, 0] + p_ref[c, 1] + hs_ref[c]
        x = jnp.maximum(dis * agg + b_ref[c], 0.0)
        acc = acc + jnp.dot(x, w_ref[c], preferred_element_type=_F32,
                            precision=_HI_PREC)
    out_ref[...] = acc + bfc_ref[0, 0]


def _tc3(p2, hs2, dis, b2r, Wfcr, bfcr):
    return pl.pallas_call(
        _tc3_body,
        grid=(GRID,),
        in_specs=[
            pl.BlockSpec((4, NC, RB, CW), lambda i: (0, 0, i, 0)),
            pl.BlockSpec((4, RB, CW), lambda i: (0, i, 0)),
            pl.BlockSpec((RB, 1), lambda i: (i, 0)),
            pl.BlockSpec((4, 1, CW), lambda i: (0, 0, 0)),
            pl.BlockSpec((4, CW, 1), lambda i: (0, 0, 0)),
            pl.BlockSpec((1, 1), lambda i: (0, 0)),
        ],
        out_specs=pl.BlockSpec((RB, 1), lambda i: (i, 0)),
        out_shape=jax.ShapeDtypeStruct((N, 1), _F32),
    )(p2, hs2, dis, b2r, Wfcr, bfcr)


def kernel(edge_index, emb, W1, b1, W2, b2, Wfc, bfc):
    src_b = edge_index[0].reshape(NBT, BL)
    dst_b = edge_index[1].reshape(NBT, BL)
    dst_f = edge_index[1]
    zeros_n = jnp.zeros((NPAD,), _F32)
    zeros_t = jnp.zeros((BL, CW), _F32)

    deg_parts = _deg_kernel(dst_f, zeros_n)
    degT = deg_parts.T
    dis, hs1 = _tc1(degT, emb, W1)
    p1 = _agg8(src_b, dst_b, hs1, zeros_t)
    hs2 = _tc2(p1, hs1, dis, b1.reshape(8, 1, CW), W2)
    p2 = _agg4(src_b, dst_b, hs2, zeros_t)
    return _tc3(p2, hs2, dis, b2.reshape(4, 1, CW),
                Wfc.reshape(4, CW, 1), bfc.reshape(1, 1))


# speedup vs baseline: 30.5895x; 3.3530x over previous
import functools

import jax
import jax.numpy as jnp
from jax import lax
from jax.experimental import pallas as pl
from jax.experimental.pallas import tpu as pltpu
from jax.experimental.pallas import tpu_sc as plsc

N = 50000
E = 800000
NC, NS = 2, 16
NW = NC * NS
BL = 128
NBT = E // BL
NB_HI = 196
NB_LO = 195
HI = NBT - NW * NB_LO
CW = 32
GS = 5
NGRP = NB_LO // GS
NPAD = 50048
SLICE = NPAD // NS
ZBR = 136
ZB = SLICE // ZBR
_MESH = dict(core_axis_name="c", subcore_axis_name="s", num_cores=NC,
             num_subcores=NS)
_F32 = jnp.float32
_SC_PARAMS = pltpu.CompilerParams(needs_layout_passes=False,
                                  use_tc_tiling_on_sc=False)


def _tile_range(wid):
    row0 = NB_LO * wid + jnp.minimum(wid, HI)
    nblk = jnp.where(wid < HI, NB_HI, NB_LO)
    return row0, nblk


@functools.partial(
    pl.kernel,
    out_type=jax.ShapeDtypeStruct((NW, NPAD), _F32),
    mesh=plsc.VectorSubcoreMesh(**_MESH),
    compiler_params=_SC_PARAMS,
    scratch_types=[
        pltpu.VMEM((NB_HI * BL,), jnp.int32),
        pltpu.VMEM((NPAD,), _F32),
    ],
)
def _deg_kernel(dst_hbm, zeros_hbm, out_hbm, dst_v, deg_v):
    wid = lax.axis_index("c") * NS + lax.axis_index("s")
    row0, nblk = _tile_range(wid)
    e0 = row0 * BL

    @pl.when(wid < HI)
    def _():
        pltpu.sync_copy(dst_hbm.at[pl.ds(e0, NB_HI * BL)], dst_v)

    @pl.when(wid >= HI)
    def _():
        pltpu.sync_copy(dst_hbm.at[pl.ds(e0, NB_LO * BL)],
                        dst_v.at[pl.ds(0, NB_LO * BL)])

    pltpu.sync_copy(zeros_hbm, deg_v)
    ones = jnp.ones((16,), _F32)

    def body(q, carry):
        idx = dst_v[pl.ds(q * 16, 16)]
        plsc.addupdate_scatter(deg_v, [idx], ones)
        return carry

    lax.fori_loop(0, nblk * (BL // 16), body, 0)
    pltpu.sync_copy(deg_v, out_hbm.at[wid])


def _make_agg_kernel(n_chunks):
    @functools.partial(
        pl.kernel,
        out_type=jax.ShapeDtypeStruct((n_chunks, NC, NPAD, CW), _F32),
        mesh=plsc.VectorSubcoreMesh(**_MESH),
        compiler_params=_SC_PARAMS,
        scratch_types=[
            pltpu.VMEM((2, GS, BL), jnp.int32),
            pltpu.VMEM((2, GS, BL), jnp.int32),
            pltpu.VMEM((GS, BL, CW), _F32),
            pltpu.VMEM((ZBR, CW), _F32),
            pltpu.VMEM_SHARED((NPAD, CW), _F32),
            pltpu.SemaphoreType.DMA((2,)),
            pltpu.SemaphoreType.DMA((GS,)),
            pltpu.SemaphoreType.DMA((GS,)),
        ],
    )
    def agg_kernel(src_hbm, dst_hbm, tbl_hbm, zeros_hbm, out_hbm,
                   isrc, idst, rows_v, zbuf, agg_sh, isem, gsem, ssem):
        cid = lax.axis_index("c")
        sid = lax.axis_index("s")
        wid = cid * NS + sid
        base = sid * SLICE
        row0, _ = _tile_range(wid)

        pltpu.sync_copy(zeros_hbm, zbuf)

        def idx_start(g, par):
            pltpu.async_copy(src_hbm.at[pl.ds(row0 + g * GS, GS)],
                             isrc.at[par], isem.at[par])
            pltpu.async_copy(dst_hbm.at[pl.ds(row0 + g * GS, GS)],
                             idst.at[par], isem.at[par])

        def idx_wait(g, par):
            pltpu.make_async_copy(src_hbm.at[pl.ds(row0 + g * GS, GS)],
                                  isrc.at[par], isem.at[par]).wait()
            pltpu.make_async_copy(dst_hbm.at[pl.ds(row0 + g * GS, GS)],
                                  idst.at[par], isem.at[par]).wait()

        idx_start(0, 0)

        for t in range(n_chunks):
            def zbody(z, carry):
                pltpu.sync_copy(zbuf, agg_sh.at[pl.ds(base + z * ZBR, ZBR)])
                return carry

            lax.fori_loop(0, ZB, zbody, 0)
            plsc.subcore_barrier()

            tbl = tbl_hbm.at[t]

            def grp(g, carry):
                par = lax.rem(g, 2)
                idx_wait(g, par)

                for b in range(GS):
                    @pl.when(g > 0)
                    def _():
                        pltpu.make_async_copy(
                            rows_v.at[b], agg_sh.at[idst.at[par].at[b]],
                            ssem.at[b]).wait()

                    pltpu.async_copy(tbl.at[isrc.at[par].at[b]],
                                     rows_v.at[b], gsem.at[b])

                @pl.when(g + 1 < NGRP)
                def _():
                    idx_start(g + 1, 1 - par)

                for b in range(GS):
                    pltpu.make_async_copy(tbl.at[isrc.at[par].at[b]],
                                          rows_v.at[b], gsem.at[b]).wait()
                    pltpu.async_copy(rows_v.at[b],
                                     agg_sh.at[idst.at[par].at[b]],
                                     ssem.at[b], add=True)
                return carry

            lax.fori_loop(0, NGRP, grp, 0)
            for b in range(GS):
                pltpu.make_async_copy(rows_v.at[b],
                                      agg_sh.at[idst.at[0].at[b]],
                                      ssem.at[b]).wait()

            @pl.when(wid < HI)
            def _():
                pltpu.sync_copy(src_hbm.at[pl.ds(row0 + NB_LO, 1)],
                                isrc.at[0].at[pl.ds(0, 1)])
                pltpu.sync_copy(dst_hbm.at[pl.ds(row0 + NB_LO, 1)],
                                idst.at[0].at[pl.ds(0, 1)])
                pltpu.async_copy(tbl.at[isrc.at[0].at[0]], rows_v.at[0],
                                 gsem.at[0])
                pltpu.make_async_copy(tbl.at[isrc.at[0].at[0]], rows_v.at[0],
                                      gsem.at[0]).wait()
                pltpu.sync_copy(rows_v.at[0], agg_sh.at[idst.at[0].at[0]],
                                add=True)

            plsc.subcore_barrier()

            pltpu.sync_copy(agg_sh.at[pl.ds(base, SLICE)],
                            out_hbm.at[t, cid, pl.ds(base, SLICE)])
            plsc.subcore_barrier()

            if t + 1 < n_chunks:
                idx_start(0, 0)

    return agg_kernel


_agg_l1 = _make_agg_kernel(4)
_agg_l2 = _make_agg_kernel(2)


_HI_PREC = jax.lax.Precision.HIGHEST
XB = 256
XROWS = N // 4
DROWS = NPAD // 4
TGRID = 49


def _dot(a, b):
    return jnp.dot(a, b, preferred_element_type=_F32, precision=_HI_PREC)


def _tc1_body(deg_ref, emb_ref, w_ref, dis_ref, hs_ref):
    d4 = jnp.sum(deg_ref[...], axis=2) + 1.0
    rep32 = jnp.repeat(jnp.eye(4, dtype=_F32), CW, axis=1)
    dis32 = _dot(jax.lax.rsqrt(d4), rep32)
    hg = [_dot(emb_ref[:, g * 64:(g + 1) * 64], w_ref[...])
          for g in range(4)]
    dis_ref[...] = dis32
    for t in range(4):
        hs_ref[t] = dis32 * jnp.concatenate(
            [hg[g][:, t * CW:(t + 1) * CW] for g in range(4)], axis=1)


def _tc1(degP, embp, W1):
    return pl.pallas_call(
        _tc1_body,
        grid=(TGRID,),
        in_specs=[
            pl.BlockSpec((XB, 4, NW), lambda i: (i, 0, 0)),
            pl.BlockSpec((XB, 256), lambda i: (i, 0)),
            pl.BlockSpec((64, 128), lambda i: (0, 0)),
        ],
        out_specs=[
            pl.BlockSpec((XB, 128), lambda i: (i, 0)),
            pl.BlockSpec((4, XB, 128), lambda i: (0, i, 0)),
        ],
        out_shape=[
            jax.ShapeDtypeStruct((XROWS, 128), _F32),
            jax.ShapeDtypeStruct((4, XROWS, 128), _F32),
        ],
    )(degP, embp, W1)


def _tc2_body(p_ref, hs_ref, dis_ref, b_ref, w_ref, out_ref):
    dis32 = dis_ref[...]
    xs = []
    for t in range(4):
        agg = p_ref[t, 0] + p_ref[t, 1] + hs_ref[t]
        xs.append(dis32 * jnp.maximum(dis32 * agg + b_ref[t], 0.0))
    hg = []
    for g in range(4):
        xg = jnp.concatenate([xs[t][:, g * CW:(g + 1) * CW] for t in range(4)],
                             axis=1)
        hg.append(_dot(xg, w_ref[...]))
    for d in range(2):
        out_ref[d] = jnp.concatenate(
            [hg[g][:, d * CW:(d + 1) * CW] for g in range(4)], axis=1)


def _tc2(pp1, hs1p, dis32, b1p, W2):
    return pl.pallas_call(
        _tc2_body,
        grid=(TGRID,),
        in_specs=[
            pl.BlockSpec((4, NC, XB, 128), lambda i: (0, 0, i, 0)),
            pl.BlockSpec((4, XB, 128), lambda i: (0, i, 0)),
            pl.BlockSpec((XB, 128), lambda i: (i, 0)),
            pl.BlockSpec((4, 1, 128), lambda i: (0, 0, 0)),
            pl.BlockSpec((128, 64), lambda i: (0, 0)),
        ],
        out_specs=pl.BlockSpec((2, XB, 128), lambda i: (0, i, 0)),
        out_shape=jax.ShapeDtypeStruct((2, XROWS, 128), _F32),
    )(pp1, hs1p, dis32, b1p, W2)


def _tc3_body(p_ref, hs_ref, dis_ref, b_ref, w_ref, bfc_ref, out_ref):
    dis32 = dis_ref[...]
    xs = []
    for d in range(2):
        agg = p_ref[d, 0] + p_ref[d, 1] + hs_ref[d]
        xs.append(jnp.maximum(dis32 * agg + b_ref[d], 0.0))
    cols = []
    for g in range(4):
        xg = jnp.concatenate([xs[d][:, g * CW:(g + 1) * CW] for d in range(2)],
                             axis=1)
        cols.append(_dot(xg, w_ref[...]))
    out_ref[...] = jnp.concatenate(cols, axis=1) + bfc_ref[0, 0]


def _tc3(pp2, hs2p, dis32, b2p, Wfc, bfcr):
    return pl.pallas_call(
        _tc3_body,
        grid=(TGRID,),
        in_specs=[
            pl.BlockSpec((2, NC, XB, 128), lambda i: (0, 0, i, 0)),
            pl.BlockSpec((2, XB, 128), lambda i: (0, i, 0)),
            pl.BlockSpec((XB, 128), lambda i: (i, 0)),
            pl.BlockSpec((2, 1, 128), lambda i: (0, 0, 0)),
            pl.BlockSpec((64, 1), lambda i: (0, 0)),
            pl.BlockSpec((1, 1), lambda i: (0, 0)),
        ],
        out_specs=pl.BlockSpec((XB, 4), lambda i: (i, 0)),
        out_shape=jax.ShapeDtypeStruct((XROWS, 4), _F32),
    )(pp2, hs2p, dis32, b2p, Wfc, bfcr)


def kernel(edge_index, emb, W1, b1, W2, b2, Wfc, bfc):
    src_b = edge_index[0].reshape(NBT, BL)
    dst_b = edge_index[1].reshape(NBT, BL)
    dst_f = edge_index[1]
    zeros_n = jnp.zeros((NPAD,), _F32)
    zeros_t = jnp.zeros((ZBR, CW), _F32)
    b1p = jnp.tile(b1.reshape(4, 1, CW), (1, 1, 4)).reshape(4, 1, 128)
    b2p = jnp.tile(b2.reshape(2, 1, CW), (1, 1, 4)).reshape(2, 1, 128)

    deg_parts = _deg_kernel(dst_f, zeros_n)
    degP = deg_parts.T.reshape(DROWS, 4, NW)
    embp = emb.reshape(XROWS, 256)
    dis32, hs1p = _tc1(degP, embp, W1)
    hs1_sc = hs1p.reshape(4, N, CW)
    p1 = _agg_l1(src_b, dst_b, hs1_sc, zeros_t)
    pp1 = p1.reshape(4, NC, DROWS, 128)
    hs2p = _tc2(pp1, hs1p, dis32, b1p, W2)
    hs2_sc = hs2p.reshape(2, N, CW)
    p2 = _agg_l2(src_b, dst_b, hs2_sc, zeros_t)
    pp2 = p2.reshape(2, NC, DROWS, 128)
    out4 = _tc3(pp2, hs2p, dis32, b2p, Wfc, bfc.reshape(1, 1))
    return out4.reshape(N, 1)


# speedup vs baseline: 31.9930x; 1.0459x over previous
import functools

import jax
import jax.numpy as jnp
from jax import lax
from jax.experimental import pallas as pl
from jax.experimental.pallas import tpu as pltpu
from jax.experimental.pallas import tpu_sc as plsc

N = 50000
E = 800000
NC, NS = 2, 16
NW = NC * NS
BL = 128
NBT = E // BL
NB_HI = 196
NB_LO = 195
HI = NBT - NW * NB_LO
CW = 32
GS = 5
NGRP = NB_LO // GS
NPAD = 50048
SLICE = NPAD // NS
ZBR = 136
ZB = SLICE // ZBR
_MESH = dict(core_axis_name="c", subcore_axis_name="s", num_cores=NC,
             num_subcores=NS)
_F32 = jnp.float32
_SC_PARAMS = pltpu.CompilerParams(needs_layout_passes=False,
                                  use_tc_tiling_on_sc=False)


def _tile_range(wid):
    row0 = NB_LO * wid + jnp.minimum(wid, HI)
    nblk = jnp.where(wid < HI, NB_HI, NB_LO)
    return row0, nblk


@functools.partial(
    pl.kernel,
    out_type=jax.ShapeDtypeStruct((NW, NPAD), _F32),
    mesh=plsc.VectorSubcoreMesh(**_MESH),
    compiler_params=_SC_PARAMS,
    scratch_types=[
        pltpu.VMEM((NB_HI * BL,), jnp.int32),
        pltpu.VMEM((NPAD,), _F32),
    ],
)
def _deg_kernel(dst_hbm, zeros_hbm, out_hbm, dst_v, deg_v):
    wid = lax.axis_index("c") * NS + lax.axis_index("s")
    row0, nblk = _tile_range(wid)
    e0 = row0 * BL

    @pl.when(wid < HI)
    def _():
        pltpu.sync_copy(dst_hbm.at[pl.ds(e0, NB_HI * BL)], dst_v)

    @pl.when(wid >= HI)
    def _():
        pltpu.sync_copy(dst_hbm.at[pl.ds(e0, NB_LO * BL)],
                        dst_v.at[pl.ds(0, NB_LO * BL)])

    pltpu.sync_copy(zeros_hbm, deg_v)
    ones = jnp.ones((16,), _F32)

    def body(q, carry):
        idx = dst_v[pl.ds(q * 16, 16)]
        plsc.addupdate_scatter(deg_v, [idx], ones)
        return carry

    lax.fori_loop(0, nblk * (BL // 16), body, 0)
    pltpu.sync_copy(deg_v, out_hbm.at[wid])


def _make_agg_kernel(n_chunks):
    @functools.partial(
        pl.kernel,
        out_type=jax.ShapeDtypeStruct((n_chunks, NC, NPAD, CW), _F32),
        mesh=plsc.VectorSubcoreMesh(**_MESH),
        compiler_params=_SC_PARAMS,
        scratch_types=[
            pltpu.VMEM((2, GS, BL), jnp.int32),
            pltpu.VMEM((2, GS, BL), jnp.int32),
            pltpu.VMEM((GS, BL, CW), _F32),
            pltpu.VMEM((ZBR, CW), _F32),
            pltpu.VMEM_SHARED((NPAD, CW), _F32),
            pltpu.SemaphoreType.DMA((2,)),
            pltpu.SemaphoreType.DMA((GS,)),
            pltpu.SemaphoreType.DMA((GS,)),
        ],
    )
    def agg_kernel(src_hbm, dst_hbm, tbl_hbm, zeros_hbm, out_hbm,
                   isrc, idst, rows_v, zbuf, agg_sh, isem, gsem, ssem):
        cid = lax.axis_index("c")
        sid = lax.axis_index("s")
        wid = cid * NS + sid
        base = sid * SLICE
        row0, _ = _tile_range(wid)

        pltpu.sync_copy(zeros_hbm, zbuf)

        def idx_start(g, par):
            pltpu.async_copy(src_hbm.at[pl.ds(row0 + g * GS, GS)],
                             isrc.at[par], isem.at[par])
            pltpu.async_copy(dst_hbm.at[pl.ds(row0 + g * GS, GS)],
                             idst.at[par], isem.at[par])

        def idx_wait(g, par):
            pltpu.make_async_copy(src_hbm.at[pl.ds(row0 + g * GS, GS)],
                                  isrc.at[par], isem.at[par]).wait()
            pltpu.make_async_copy(dst_hbm.at[pl.ds(row0 + g * GS, GS)],
                                  idst.at[par], isem.at[par]).wait()

        idx_start(0, 0)

        for t in range(n_chunks):
            def zbody(z, carry):
                pltpu.sync_copy(zbuf, agg_sh.at[pl.ds(base + z * ZBR, ZBR)])
                return carry

            lax.fori_loop(0, ZB, zbody, 0)
            plsc.subcore_barrier()

            tbl = tbl_hbm.at[t]

            def grp(g, carry):
                par = lax.rem(g, 2)
                idx_wait(g, par)

                for b in range(GS):
                    @pl.when(g > 0)
                    def _():
                        pltpu.make_async_copy(
                            rows_v.at[b], agg_sh.at[idst.at[par].at[b]],
                            ssem.at[b]).wait()

                    pltpu.async_copy(tbl.at[isrc.at[par].at[b]],
                                     rows_v.at[b], gsem.at[b])

                @pl.when(g + 1 < NGRP)
                def _():
                    idx_start(g + 1, 1 - par)

                for b in range(GS):
                    pltpu.make_async_copy(tbl.at[isrc.at[par].at[b]],
                                          rows_v.at[b], gsem.at[b]).wait()
                    pltpu.async_copy(rows_v.at[b],
                                     agg_sh.at[idst.at[par].at[b]],
                                     ssem.at[b], add=True)
                return carry

            lax.fori_loop(0, NGRP, grp, 0)
            for b in range(GS):
                pltpu.make_async_copy(rows_v.at[b],
                                      agg_sh.at[idst.at[0].at[b]],
                                      ssem.at[b]).wait()

            @pl.when(wid < HI)
            def _():
                pltpu.sync_copy(src_hbm.at[pl.ds(row0 + NB_LO, 1)],
                                isrc.at[0].at[pl.ds(0, 1)])
                pltpu.sync_copy(dst_hbm.at[pl.ds(row0 + NB_LO, 1)],
                                idst.at[0].at[pl.ds(0, 1)])
                pltpu.async_copy(tbl.at[isrc.at[0].at[0]], rows_v.at[0],
                                 gsem.at[0])
                pltpu.make_async_copy(tbl.at[isrc.at[0].at[0]], rows_v.at[0],
                                      gsem.at[0]).wait()
                pltpu.sync_copy(rows_v.at[0], agg_sh.at[idst.at[0].at[0]],
                                add=True)

            plsc.subcore_barrier()

            pltpu.sync_copy(agg_sh.at[pl.ds(base, SLICE)],
                            out_hbm.at[t, cid, pl.ds(base, SLICE)])
            plsc.subcore_barrier()

            if t + 1 < n_chunks:
                idx_start(0, 0)

    return agg_kernel


_agg_l1 = _make_agg_kernel(4)
_agg_l2 = _make_agg_kernel(2)


_HI_PREC = jax.lax.Precision.HIGHEST
XB = 512
XROWS = N // 4
DROWS = NPAD // 4
TGRID = 25


def _dot(a, b):
    return jnp.dot(a, b, preferred_element_type=_F32, precision=_HI_PREC)


def _tc1_body(deg_ref, emb_ref, w_ref, dis_ref, hs_ref):
    d4 = jnp.sum(deg_ref[...], axis=2) + 1.0
    rep32 = jnp.repeat(jnp.eye(4, dtype=_F32), CW, axis=1)
    dis32 = _dot(jax.lax.rsqrt(d4), rep32)
    hg = [_dot(emb_ref[:, g * 64:(g + 1) * 64], w_ref[...])
          for g in range(4)]
    dis_ref[...] = dis32
    for t in range(4):
        hs_ref[t] = dis32 * jnp.concatenate(
            [hg[g][:, t * CW:(t + 1) * CW] for g in range(4)], axis=1)


def _tc1(degP, embp, W1):
    return pl.pallas_call(
        _tc1_body,
        grid=(TGRID,),
        in_specs=[
            pl.BlockSpec((XB, 4, NW), lambda i: (i, 0, 0)),
            pl.BlockSpec((XB, 256), lambda i: (i, 0)),
            pl.BlockSpec((64, 128), lambda i: (0, 0)),
        ],
        out_specs=[
            pl.BlockSpec((XB, 128), lambda i: (i, 0)),
            pl.BlockSpec((4, XB, 128), lambda i: (0, i, 0)),
        ],
        out_shape=[
            jax.ShapeDtypeStruct((XROWS, 128), _F32),
            jax.ShapeDtypeStruct((4, XROWS, 128), _F32),
        ],
    )(degP, embp, W1)


def _tc2_body(p_ref, hs_ref, dis_ref, b_ref, w_ref, out_ref):
    dis32 = dis_ref[...]
    xs = []
    for t in range(4):
        agg = p_ref[t, 0] + p_ref[t, 1] + hs_ref[t]
        xs.append(dis32 * jnp.maximum(dis32 * agg + b_ref[t], 0.0))
    hg = []
    for g in range(4):
        xg = jnp.concatenate([xs[t][:, g * CW:(g + 1) * CW] for t in range(4)],
                             axis=1)
        hg.append(_dot(xg, w_ref[...]))
    for d in range(2):
        out_ref[d] = jnp.concatenate(
            [hg[g][:, d * CW:(d + 1) * CW] for g in range(4)], axis=1)


def _tc2(pp1, hs1p, dis32, b1p, W2):
    return pl.pallas_call(
        _tc2_body,
        grid=(TGRID,),
        in_specs=[
            pl.BlockSpec((4, NC, XB, 128), lambda i: (0, 0, i, 0)),
            pl.BlockSpec((4, XB, 128), lambda i: (0, i, 0)),
            pl.BlockSpec((XB, 128), lambda i: (i, 0)),
            pl.BlockSpec((4, 1, 128), lambda i: (0, 0, 0)),
            pl.BlockSpec((128, 64), lambda i: (0, 0)),
        ],
        out_specs=pl.BlockSpec((2, XB, 128), lambda i: (0, i, 0)),
        out_shape=jax.ShapeDtypeStruct((2, XROWS, 128), _F32),
    )(pp1, hs1p, dis32, b1p, W2)


def _tc3_body(p_ref, hs_ref, dis_ref, b_ref, w_ref, bfc_ref, out_ref):
    dis32 = dis_ref[...]
    xs = []
    for d in range(2):
        agg = p_ref[d, 0] + p_ref[d, 1] + hs_ref[d]
        xs.append(jnp.maximum(dis32 * agg + b_ref[d], 0.0))
    cols = []
    for g in range(4):
        xg = jnp.concatenate([xs[d][:, g * CW:(g + 1) * CW] for d in range(2)],
                             axis=1)
        cols.append(_dot(xg, w_ref[...]))
    out_ref[...] = jnp.concatenate(cols, axis=1) + bfc_ref[0, 0]


def _tc3(pp2, hs2p, dis32, b2p, Wfc, bfcr):
    return pl.pallas_call(
        _tc3_body,
        grid=(TGRID,),
        in_specs=[
            pl.BlockSpec((2, NC, XB, 128), lambda i: (0, 0, i, 0)),
            pl.BlockSpec((2, XB, 128), lambda i: (0, i, 0)),
            pl.BlockSpec((XB, 128), lambda i: (i, 0)),
            pl.BlockSpec((2, 1, 128), lambda i: (0, 0, 0)),
            pl.BlockSpec((64, 1), lambda i: (0, 0)),
            pl.BlockSpec((1, 1), lambda i: (0, 0)),
        ],
        out_specs=pl.BlockSpec((XB, 4), lambda i: (i, 0)),
        out_shape=jax.ShapeDtypeStruct((XROWS, 4), _F32),
    )(pp2, hs2p, dis32, b2p, Wfc, bfcr)


def kernel(edge_index, emb, W1, b1, W2, b2, Wfc, bfc):
    src_b = edge_index[0].reshape(NBT, BL)
    dst_b = edge_index[1].reshape(NBT, BL)
    dst_f = edge_index[1]
    zeros_n = jnp.zeros((NPAD,), _F32)
    zeros_t = jnp.zeros((ZBR, CW), _F32)
    b1p = jnp.tile(b1.reshape(4, 1, CW), (1, 1, 4)).reshape(4, 1, 128)
    b2p = jnp.tile(b2.reshape(2, 1, CW), (1, 1, 4)).reshape(2, 1, 128)

    deg_parts = _deg_kernel(dst_f, zeros_n)
    degP = deg_parts.T.reshape(DROWS, 4, NW)
    embp = emb.reshape(XROWS, 256)
    dis32, hs1p = _tc1(degP, embp, W1)
    hs1_sc = hs1p.reshape(4, N, CW)
    p1 = _agg_l1(src_b, dst_b, hs1_sc, zeros_t)
    pp1 = p1.reshape(4, NC, DROWS, 128)
    hs2p = _tc2(pp1, hs1p, dis32, b1p, W2)
    hs2_sc = hs2p.reshape(2, N, CW)
    p2 = _agg_l2(src_b, dst_b, hs2_sc, zeros_t)
    pp2 = p2.reshape(2, NC, DROWS, 128)
    out4 = _tc3(pp2, hs2p, dis32, b2p, Wfc, bfc.reshape(1, 1))
    return out4.reshape(N, 1)


# speedup vs baseline: 33.1964x; 1.0376x over previous
import functools

import jax
import jax.numpy as jnp
from jax import lax
from jax.experimental import pallas as pl
from jax.experimental.pallas import tpu as pltpu
from jax.experimental.pallas import tpu_sc as plsc

N = 50000
E = 800000
NC, NS = 2, 16
NW = NC * NS
BL = 128
NBT = E // BL
NB_HI = 196
NB_LO = 195
HI = NBT - NW * NB_LO
CW = 32
GS = 5
NGRP = NB_LO // GS
NPAD = 50048
SLICE = NPAD // NS
ZBR = 136
ZB = SLICE // ZBR
_MESH = dict(core_axis_name="c", subcore_axis_name="s", num_cores=NC,
             num_subcores=NS)
_F32 = jnp.float32
_SC_PARAMS = pltpu.CompilerParams(needs_layout_passes=False,
                                  use_tc_tiling_on_sc=False)


def _tile_range(wid):
    row0 = NB_LO * wid + jnp.minimum(wid, HI)
    nblk = jnp.where(wid < HI, NB_HI, NB_LO)
    return row0, nblk


@functools.partial(
    pl.kernel,
    out_type=jax.ShapeDtypeStruct((NW, NPAD), _F32),
    mesh=plsc.VectorSubcoreMesh(**_MESH),
    compiler_params=_SC_PARAMS,
    scratch_types=[
        pltpu.VMEM((NB_HI * BL,), jnp.int32),
        pltpu.VMEM((NPAD,), _F32),
    ],
)
def _deg_kernel(dst_hbm, zeros_hbm, out_hbm, dst_v, deg_v):
    wid = lax.axis_index("c") * NS + lax.axis_index("s")
    row0, nblk = _tile_range(wid)
    e0 = row0 * BL

    @pl.when(wid < HI)
    def _():
        pltpu.sync_copy(dst_hbm.at[pl.ds(e0, NB_HI * BL)], dst_v)

    @pl.when(wid >= HI)
    def _():
        pltpu.sync_copy(dst_hbm.at[pl.ds(e0, NB_LO * BL)],
                        dst_v.at[pl.ds(0, NB_LO * BL)])

    pltpu.sync_copy(zeros_hbm, deg_v)
    ones = jnp.ones((16,), _F32)

    def body(q, carry):
        idx = dst_v[pl.ds(q * 16, 16)]
        plsc.addupdate_scatter(deg_v, [idx], ones)
        return carry

    lax.fori_loop(0, nblk * (BL // 16), body, 0)
    pltpu.sync_copy(deg_v, out_hbm.at[wid])


def _make_agg_kernel(n_chunks):
    @functools.partial(
        pl.kernel,
        out_type=jax.ShapeDtypeStruct((n_chunks, NC, NPAD, CW), _F32),
        mesh=plsc.VectorSubcoreMesh(**_MESH),
        compiler_params=_SC_PARAMS,
        scratch_types=[
            pltpu.VMEM((2, GS, BL), jnp.int32),
            pltpu.VMEM((2, GS, BL), jnp.int32),
            pltpu.VMEM((GS, BL, CW), _F32),
            pltpu.VMEM((ZBR, CW), _F32),
            pltpu.VMEM_SHARED((NPAD, CW), _F32),
            pltpu.SemaphoreType.DMA((2,)),
            pltpu.SemaphoreType.DMA((GS,)),
            pltpu.SemaphoreType.DMA((GS,)),
        ],
    )
    def agg_kernel(src_hbm, dst_hbm, tbl_hbm, zeros_hbm, out_hbm,
                   isrc, idst, rows_v, zbuf, agg_sh, isem, gsem, ssem):
        cid = lax.axis_index("c")
        sid = lax.axis_index("s")
        wid = cid * NS + sid
        base = sid * SLICE
        row0, _ = _tile_range(wid)

        pltpu.sync_copy(zeros_hbm, zbuf)

        def idx_start(g, par):
            pltpu.async_copy(src_hbm.at[pl.ds(row0 + g * GS, GS)],
                             isrc.at[par], isem.at[par])
            pltpu.async_copy(dst_hbm.at[pl.ds(row0 + g * GS, GS)],
                             idst.at[par], isem.at[par])

        def idx_wait(g, par):
            pltpu.make_async_copy(src_hbm.at[pl.ds(row0 + g * GS, GS)],
                                  isrc.at[par], isem.at[par]).wait()
            pltpu.make_async_copy(dst_hbm.at[pl.ds(row0 + g * GS, GS)],
                                  idst.at[par], isem.at[par]).wait()

        idx_start(0, 0)

        for t in range(n_chunks):
            def zbody(z, carry):
                pltpu.sync_copy(zbuf, agg_sh.at[pl.ds(base + z * ZBR, ZBR)])
                return carry

            lax.fori_loop(0, ZB, zbody, 0)
            plsc.subcore_barrier()

            tbl = tbl_hbm.at[t]

            def grp(g, carry):
                par = lax.rem(g, 2)
                idx_wait(g, par)

                for b in range(GS):
                    @pl.when(g > 0)
                    def _():
                        pltpu.make_async_copy(
                            rows_v.at[b], agg_sh.at[idst.at[par].at[b]],
                            ssem.at[b]).wait()

                    pltpu.async_copy(tbl.at[isrc.at[par].at[b]],
                                     rows_v.at[b], gsem.at[b])

                @pl.when(g + 1 < NGRP)
                def _():
                    idx_start(g + 1, 1 - par)

                for b in range(GS):
                    pltpu.make_async_copy(tbl.at[isrc.at[par].at[b]],
                                          rows_v.at[b], gsem.at[b]).wait()
                    pltpu.async_copy(rows_v.at[b],
                                     agg_sh.at[idst.at[par].at[b]],
                                     ssem.at[b], add=True)
                return carry

            lax.fori_loop(0, NGRP, grp, 0)
            for b in range(GS):
                pltpu.make_async_copy(rows_v.at[b],
                                      agg_sh.at[idst.at[0].at[b]],
                                      ssem.at[b]).wait()

            @pl.when(wid < HI)
            def _():
                pltpu.sync_copy(src_hbm.at[pl.ds(row0 + NB_LO, 1)],
                                isrc.at[0].at[pl.ds(0, 1)])
                pltpu.sync_copy(dst_hbm.at[pl.ds(row0 + NB_LO, 1)],
                                idst.at[0].at[pl.ds(0, 1)])
                pltpu.async_copy(tbl.at[isrc.at[0].at[0]], rows_v.at[0],
                                 gsem.at[0])
                pltpu.make_async_copy(tbl.at[isrc.at[0].at[0]], rows_v.at[0],
                                      gsem.at[0]).wait()
                pltpu.sync_copy(rows_v.at[0], agg_sh.at[idst.at[0].at[0]],
                                add=True)

            plsc.subcore_barrier()

            pltpu.sync_copy(agg_sh.at[pl.ds(base, SLICE)],
                            out_hbm.at[t, cid, pl.ds(base, SLICE)])
            plsc.subcore_barrier()

            if t + 1 < n_chunks:
                idx_start(0, 0)

    return agg_kernel


_agg_l1 = _make_agg_kernel(4)
_agg_l2 = _make_agg_kernel(2)


_HI_PREC = jax.lax.Precision.HIGHEST
XB = 512
XROWS = N // 4
DROWS = NPAD // 4
TGRID = 25


def _dot(a, b):
    return jnp.dot(a, b, preferred_element_type=_F32)


def _tc1_body(deg_ref, emb_ref, w_ref, dis_ref, hs_ref):
    d4 = jnp.sum(deg_ref[...], axis=2) + 1.0
    rep32 = jnp.repeat(jnp.eye(4, dtype=_F32), CW, axis=1)
    dis32 = jnp.dot(jax.lax.rsqrt(d4), rep32,
                    preferred_element_type=_F32,
                    precision=_HI_PREC)
    hg = [_dot(emb_ref[:, g * 64:(g + 1) * 64], w_ref[...])
          for g in range(4)]
    dis_ref[...] = dis32
    for t in range(4):
        hs_ref[t] = dis32 * jnp.concatenate(
            [hg[g][:, t * CW:(t + 1) * CW] for g in range(4)], axis=1)


def _tc1(degP, embp, W1):
    return pl.pallas_call(
        _tc1_body,
        grid=(TGRID,),
        in_specs=[
            pl.BlockSpec((XB, 4, NW), lambda i: (i, 0, 0)),
            pl.BlockSpec((XB, 256), lambda i: (i, 0)),
            pl.BlockSpec((64, 128), lambda i: (0, 0)),
        ],
        out_specs=[
            pl.BlockSpec((XB, 128), lambda i: (i, 0)),
            pl.BlockSpec((4, XB, 128), lambda i: (0, i, 0)),
        ],
        out_shape=[
            jax.ShapeDtypeStruct((XROWS, 128), _F32),
            jax.ShapeDtypeStruct((4, XROWS, 128), _F32),
        ],
    )(degP, embp, W1)


def _tc2_body(p_ref, hs_ref, dis_ref, b_ref, w_ref, out_ref):
    dis32 = dis_ref[...]
    xs = []
    for t in range(4):
        agg = p_ref[t, 0] + p_ref[t, 1] + hs_ref[t]
        xs.append(dis32 * jnp.maximum(dis32 * agg + b_ref[t], 0.0))
    hg = []
    for g in range(4):
        xg = jnp.concatenate([xs[t][:, g * CW:(g + 1) * CW] for t in range(4)],
                             axis=1)
        hg.append(_dot(xg, w_ref[...]))
    for d in range(2):
        out_ref[d] = jnp.concatenate(
            [hg[g][:, d * CW:(d + 1) * CW] for g in range(4)], axis=1)


def _tc2(pp1, hs1p, dis32, b1p, W2):
    return pl.pallas_call(
        _tc2_body,
        grid=(TGRID,),
        in_specs=[
            pl.BlockSpec((4, NC, XB, 128), lambda i: (0, 0, i, 0)),
            pl.BlockSpec((4, XB, 128), lambda i: (0, i, 0)),
            pl.BlockSpec((XB, 128), lambda i: (i, 0)),
            pl.BlockSpec((4, 1, 128), lambda i: (0, 0, 0)),
            pl.BlockSpec((128, 64), lambda i: (0, 0)),
        ],
        out_specs=pl.BlockSpec((2, XB, 128), lambda i: (0, i, 0)),
        out_shape=jax.ShapeDtypeStruct((2, XROWS, 128), _F32),
    )(pp1, hs1p, dis32, b1p, W2)


def _tc3_body(p_ref, hs_ref, dis_ref, b_ref, w_ref, bfc_ref, out_ref):
    dis32 = dis_ref[...]
    xs = []
    for d in range(2):
        agg = p_ref[d, 0] + p_ref[d, 1] + hs_ref[d]
        xs.append(jnp.maximum(dis32 * agg + b_ref[d], 0.0))
    cols = []
    for g in range(4):
        xg = jnp.concatenate([xs[d][:, g * CW:(g + 1) * CW] for d in range(2)],
                             axis=1)
        cols.append(_dot(xg, w_ref[...]))
    out_ref[...] = jnp.concatenate(cols, axis=1) + bfc_ref[0, 0]


def _tc3(pp2, hs2p, dis32, b2p, Wfc, bfcr):
    return pl.pallas_call(
        _tc3_body,
        grid=(TGRID,),
        in_specs=[
            pl.BlockSpec((2, NC, XB, 128), lambda i: (0, 0, i, 0)),
            pl.BlockSpec((2, XB, 128), lambda i: (0, i, 0)),
            pl.BlockSpec((XB, 128), lambda i: (i, 0)),
            pl.BlockSpec((2, 1, 128), lambda i: (0, 0, 0)),
            pl.BlockSpec((64, 1), lambda i: (0, 0)),
            pl.BlockSpec((1, 1), lambda i: (0, 0)),
        ],
        out_specs=pl.BlockSpec((XB, 4), lambda i: (i, 0)),
        out_shape=jax.ShapeDtypeStruct((XROWS, 4), _F32),
    )(pp2, hs2p, dis32, b2p, Wfc, bfcr)


def kernel(edge_index, emb, W1, b1, W2, b2, Wfc, bfc):
    src_b = edge_index[0].reshape(NBT, BL)
    dst_b = edge_index[1].reshape(NBT, BL)
    dst_f = edge_index[1]
    zeros_n = jnp.zeros((NPAD,), _F32)
    zeros_t = jnp.zeros((ZBR, CW), _F32)
    b1p = jnp.tile(b1.reshape(4, 1, CW), (1, 1, 4)).reshape(4, 1, 128)
    b2p = jnp.tile(b2.reshape(2, 1, CW), (1, 1, 4)).reshape(2, 1, 128)

    deg_parts = _deg_kernel(dst_f, zeros_n)
    degP = deg_parts.T.reshape(DROWS, 4, NW)
    embp = emb.reshape(XROWS, 256)
    dis32, hs1p = _tc1(degP, embp, W1)
    hs1_sc = hs1p.reshape(4, N, CW)
    p1 = _agg_l1(src_b, dst_b, hs1_sc, zeros_t)
    pp1 = p1.reshape(4, NC, DROWS, 128)
    hs2p = _tc2(pp1, hs1p, dis32, b1p, W2)
    hs2_sc = hs2p.reshape(2, N, CW)
    p2 = _agg_l2(src_b, dst_b, hs2_sc, zeros_t)
    pp2 = p2.reshape(2, NC, DROWS, 128)
    out4 = _tc3(pp2, hs2p, dis32, b2p, Wfc, bfc.reshape(1, 1))
    return out4.reshape(N, 1)
